# baseline (speedup 1.0000x reference)
import functools

import jax
import jax.numpy as jnp
from jax import lax
from jax.experimental import pallas as pl
from jax.experimental.pallas import tpu as pltpu
from jax.experimental.pallas import tpu_sc as plsc

_B = 16384
_K = 64
_NW = 32
_BPW = _B // _NW
_CH = 128
_NCH = _BPW // _CH


def _sc_scores(sU, sI, sJ, betaI, gammaU, gammaI):
    mesh = plsc.VectorSubcoreMesh(core_axis_name="c", subcore_axis_name="s")

    @functools.partial(
        pl.kernel,
        out_type=jax.ShapeDtypeStruct((_B,), jnp.float32),
        mesh=mesh,
        compiler_params=pltpu.CompilerParams(
            needs_layout_passes=False, use_tc_tiling_on_sc=False),
        scratch_types=[
            pltpu.VMEM((_NCH, _CH), jnp.int32),
            pltpu.VMEM((_NCH, _CH), jnp.int32),
            pltpu.VMEM((_NCH, _CH), jnp.int32),
            pltpu.VMEM((_BPW, _K), jnp.float32),
            pltpu.VMEM((_BPW, _K), jnp.float32),
            pltpu.VMEM((_BPW, _K), jnp.float32),
            pltpu.VMEM((_BPW,), jnp.float32),
            pltpu.VMEM((_BPW,), jnp.float32),
            pltpu.VMEM((_BPW,), jnp.float32),
            pltpu.SemaphoreType.DMA,
        ],
    )
    def body(sU_h, sI_h, sJ_h, betaI_h, gU_h, gI_h, out_h,
             idx_u, idx_i, idx_j, gu, gi, gj, bi, bj, xo, sem):
        wid = lax.axis_index("s") * 2 + lax.axis_index("c")
        rbase = wid * _NCH
        pltpu.sync_copy(sU_h.at[pl.ds(rbase, _NCH)], idx_u)
        pltpu.sync_copy(sI_h.at[pl.ds(rbase, _NCH)], idx_i)
        pltpu.sync_copy(sJ_h.at[pl.ds(rbase, _NCH)], idx_j)

        cps = []
        for c in range(_NCH):
            sl = pl.ds(c * _CH, _CH)
            cps.append(pltpu.async_copy(gU_h.at[idx_u.at[c]], gu.at[sl], sem))
            cps.append(pltpu.async_copy(gI_h.at[idx_i.at[c]], gi.at[sl], sem))
            cps.append(pltpu.async_copy(gI_h.at[idx_j.at[c]], gj.at[sl], sem))
            cps.append(pltpu.async_copy(betaI_h.at[idx_i.at[c]], bi.at[sl], sem))
            cps.append(pltpu.async_copy(betaI_h.at[idx_j.at[c]], bj.at[sl], sem))
        for cp in cps:
            cp.wait()

        lane = lax.broadcasted_iota(jnp.int32, (16,), 0)

        def group(g, carry):
            row = g * 16 + lane
            acc = bi[pl.ds(g * 16, 16)] - bj[pl.ds(g * 16, 16)]
            for k in range(_K):
                kv = jnp.full((16,), k, jnp.int32)
                cu = plsc.load_gather(gu, [row, kv])
                ci = plsc.load_gather(gi, [row, kv])
                cj = plsc.load_gather(gj, [row, kv])
                acc = acc + cu * (ci - cj)
            xo[pl.ds(g * 16, 16)] = acc
            return carry

        lax.fori_loop(0, _BPW // 16, group, 0)
        pltpu.sync_copy(xo, out_h.at[pl.ds(wid * _BPW, _BPW)])

    return body(sU, sI, sJ, betaI, gammaU, gammaI)


def _tc_loss(x2d):
    def body(x_ref, o_ref):
        v = x_ref[...]
        sp = jnp.maximum(-v, 0.0) + jnp.log1p(jnp.exp(-jnp.abs(v)))
        o_ref[...] = (jnp.sum(sp) * (1.0 / _B)).reshape(1, 1)

    return pl.pallas_call(
        body,
        out_shape=jax.ShapeDtypeStruct((1, 1), jnp.float32),
    )(x2d)


def kernel(sampleU, sampleI, sampleJ, betaI, gammaU, gammaI):
    sU = sampleU.reshape(128, 128)
    sI = sampleI.reshape(128, 128)
    sJ = sampleJ.reshape(128, 128)
    x = _sc_scores(sU, sI, sJ, betaI, gammaU, gammaI)
    return _tc_loss(x.reshape(128, 128))[0, 0]


# speedup vs baseline: 1.5539x; 1.5539x over previous
import functools

import jax
import jax.numpy as jnp
from jax import lax
from jax.experimental import pallas as pl
from jax.experimental.pallas import tpu as pltpu
from jax.experimental.pallas import tpu_sc as plsc

_B = 16384
_K = 64
_NW = 32
_BPW = _B // _NW
_CH = 128
_NCH = _BPW // _CH
_H = _BPW // 2


def _sc_scores(sU, sI, sJ, betaI, gammaU, gammaI):
    mesh = plsc.VectorSubcoreMesh(core_axis_name="c", subcore_axis_name="s")

    @functools.partial(
        pl.kernel,
        out_type=jax.ShapeDtypeStruct((_B,), jnp.float32),
        mesh=mesh,
        compiler_params=pltpu.CompilerParams(needs_layout_passes=False),
        scratch_types=[
            pltpu.VMEM((_BPW + 16,), jnp.int32),
            pltpu.VMEM((_BPW + 16,), jnp.int32),
            pltpu.VMEM((_BPW + 16,), jnp.int32),
            pltpu.VMEM((_H, _K), jnp.float32),
            pltpu.VMEM((_H, _K), jnp.float32),
            pltpu.VMEM((_H, _K), jnp.float32),
            pltpu.VMEM((_BPW,), jnp.float32),
            pltpu.VMEM((_BPW,), jnp.float32),
            pltpu.VMEM((_BPW,), jnp.float32),
            pltpu.SemaphoreType.DMA,
        ],
    )
    def body(sU_h, sI_h, sJ_h, betaI_h, gU_h, gI_h, out_h,
             idx_u, idx_i, idx_j, gu, gi, gj, bi, bj, xo, sem):
        wid = lax.axis_index("s") * 2 + lax.axis_index("c")
        base = wid * _BPW
        pltpu.sync_copy(sU_h.at[pl.ds(base, _BPW)], idx_u.at[pl.ds(0, _BPW)])
        pltpu.sync_copy(sI_h.at[pl.ds(base, _BPW)], idx_i.at[pl.ds(0, _BPW)])
        pltpu.sync_copy(sJ_h.at[pl.ds(base, _BPW)], idx_j.at[pl.ds(0, _BPW)])

        for c in range(_NCH):
            sl = pl.ds(c * _CH, _CH)
            pltpu.async_copy(betaI_h.at[idx_i.at[sl]], bi.at[sl], sem)
            pltpu.async_copy(betaI_h.at[idx_j.at[sl]], bj.at[sl], sem)

        pltpu.make_async_copy(betaI_h.at[pl.ds(0, _BPW)], bi, sem).wait()
        pltpu.make_async_copy(betaI_h.at[pl.ds(0, _BPW)], bj, sem).wait()

        lane = lax.broadcasted_iota(jnp.int32, (16,), 0)

        for h in range(2):
            off = h * _H

            def rowfetch(s, carry):
                iu = idx_u[pl.ds(off + s, 16)][0]
                ii = idx_i[pl.ds(off + s, 16)][0]
                ij = idx_j[pl.ds(off + s, 16)][0]
                dst = pl.ds(s, 1)
                pltpu.async_copy(gU_h.at[pl.ds(iu, 1)], gu.at[dst], sem)
                pltpu.async_copy(gI_h.at[pl.ds(ii, 1)], gi.at[dst], sem)
                pltpu.async_copy(gI_h.at[pl.ds(ij, 1)], gj.at[dst], sem)
                return carry

            lax.fori_loop(0, _H, rowfetch, 0)

            def rowdrain(s, carry):
                iu = idx_u[pl.ds(off + s, 16)][0]
                ii = idx_i[pl.ds(off + s, 16)][0]
                ij = idx_j[pl.ds(off + s, 16)][0]
                dst = pl.ds(s, 1)
                pltpu.make_async_copy(gU_h.at[pl.ds(iu, 1)], gu.at[dst], sem).wait()
                pltpu.make_async_copy(gI_h.at[pl.ds(ii, 1)], gi.at[dst], sem).wait()
                pltpu.make_async_copy(gI_h.at[pl.ds(ij, 1)], gj.at[dst], sem).wait()
                return carry

            lax.fori_loop(0, _H, rowdrain, 0)

            def group(g, carry):
                acc = (bi[pl.ds(off + g * 16, 16)] -
                       bj[pl.ds(off + g * 16, 16)])
                row = g * 16 + lane
                for k in range(_K):
                    kv = jnp.full((16,), k, jnp.int32)
                    cu = plsc.load_gather(gu, [row, kv])
                    ci = plsc.load_gather(gi, [row, kv])
                    cj = plsc.load_gather(gj, [row, kv])
                    acc = acc + cu * (ci - cj)
                xo[pl.ds(off + g * 16, 16)] = acc
                return carry

            lax.fori_loop(0, _H // 16, group, 0)

        pltpu.sync_copy(xo, out_h.at[pl.ds(base, _BPW)])

    return body(sU, sI, sJ, betaI, gammaU, gammaI)


def _tc_loss(x2d):
    def body(x_ref, o_ref):
        v = x_ref[...]
        sp = jnp.maximum(-v, 0.0) + jnp.log1p(jnp.exp(-jnp.abs(v)))
        o_ref[...] = (jnp.sum(sp) * (1.0 / _B)).reshape(1, 1)

    return pl.pallas_call(
        body,
        out_shape=jax.ShapeDtypeStruct((1, 1), jnp.float32),
    )(x2d)


def kernel(sampleU, sampleI, sampleJ, betaI, gammaU, gammaI):
    x = _sc_scores(sampleU, sampleI, sampleJ, betaI, gammaU, gammaI)
    return _tc_loss(x.reshape(128, 128))[0, 0]
